# baseline (speedup 1.0000x reference)
import functools

import jax
import jax.numpy as jnp
from jax.experimental import pallas as pl
from jax.experimental.pallas import tpu as pltpu


def _mish(y):
    t = jnp.exp(jnp.minimum(y, 20.0))
    u = 1.0 + t
    u2 = u * u
    return y * ((u2 - 1.0) / (u2 + 1.0))


def _conv3x3_bn_mish_kernel(x_ref, w_ref, b_ref, o_ref, *, h, w, c_in, c_out,
                            g):
    hc = h * c_in
    kwid = (h + 2) * c_in
    xws = []
    for img in range(g):
        xi = x_ref[img * hc:(img + 1) * hc, :]
        zrow = jnp.zeros((c_in, w), jnp.float32)
        xp = jnp.concatenate([zrow, xi, zrow], axis=0)
        zcol = jnp.zeros((kwid, 1), jnp.float32)
        x0 = jnp.concatenate([zcol, xp[:, :w - 1]], axis=1)
        x2 = jnp.concatenate([xp[:, 1:], zcol], axis=1)
        xws.append(jnp.concatenate([x0, xp, x2], axis=0))
    nslab = h // 8
    kslab = 10 * c_in
    for s in range(nslab):
        pieces = []
        for img in range(g):
            for kw in range(3):
                off = kw * kwid + 8 * s * c_in
                pieces.append(xws[img][off:off + kslab, :])
        rhs = jnp.concatenate(
            [jnp.concatenate(pieces[i * 3:(i + 1) * 3], axis=0)
             for i in range(g)], axis=1).astype(jnp.bfloat16)
        y = jnp.dot(w_ref[...], rhs,
                    preferred_element_type=jnp.float32) + b_ref[...]
        m = _mish(y)
        for img in range(g):
            for co in range(c_out):
                o_ref[img * c_out * h + co * h + 8 * s:
                      img * c_out * h + co * h + 8 * s + 8, :] = (
                    m[co * 8:(co + 1) * 8, img * w:(img + 1) * w])


@jax.jit
def _conv_bn_mish(x, weight, bn_gamma, bn_beta, bn_mean, bn_var):
    eps = 1e-5
    n, c_in, h, w = x.shape
    c_out = weight.shape[0]
    g = 2 if n % 2 == 0 else 1

    scale = bn_gamma / jnp.sqrt(bn_var + eps)
    bias = bn_beta - bn_mean * scale
    w_folded = weight * scale[:, None, None, None]

    ws = jnp.zeros((c_out * 8, 3 * 10 * c_in), jnp.float32)
    for dh in range(8):
        for kh in range(3):
            for kw in range(3):
                ws = ws.at[dh::8, kw * 10 * c_in + (dh + kh) * c_in:
                           kw * 10 * c_in + (dh + kh) * c_in + c_in].set(
                    w_folded[:, :, kh, kw])
    ws = ws.astype(jnp.bfloat16)
    b_col = jnp.repeat(bias, 8).reshape(c_out * 8, 1)

    xt = x.transpose(0, 2, 1, 3).reshape(n * h * c_in, w)

    out2 = pl.pallas_call(
        functools.partial(_conv3x3_bn_mish_kernel, h=h, w=w, c_in=c_in,
                          c_out=c_out, g=g),
        out_shape=jax.ShapeDtypeStruct((n * c_out * h, w), jnp.float32),
        grid_spec=pltpu.PrefetchScalarGridSpec(
            num_scalar_prefetch=0,
            grid=(n // g,),
            in_specs=[
                pl.BlockSpec((g * h * c_in, w), lambda i: (i, 0)),
                pl.BlockSpec((c_out * 8, 3 * 10 * c_in), lambda i: (0, 0)),
                pl.BlockSpec((c_out * 8, 1), lambda i: (0, 0)),
            ],
            out_specs=pl.BlockSpec((g * c_out * h, w), lambda i: (i, 0)),
        ),
        compiler_params=pltpu.CompilerParams(
            dimension_semantics=("parallel",)),
    )(xt, ws, b_col)

    return out2.reshape(n, c_out, h, w)


def kernel(x, weight, bn_gamma, bn_beta, bn_mean, bn_var):
    return _conv_bn_mish(x, weight, bn_gamma, bn_beta, bn_mean, bn_var)


# speedup vs baseline: 58.5350x; 58.5350x over previous
import functools

import jax
import jax.numpy as jnp
from jax.experimental import pallas as pl
from jax.experimental.pallas import tpu as pltpu


def _mish(y):
    t = jnp.exp(jnp.minimum(y, 20.0))
    u = 1.0 + t
    u2 = u * u
    return y * ((u2 - 1.0) / (u2 + 1.0))


def _conv3x3_bn_mish_kernel(x_ref, w_ref, b_ref, o_ref, *, h, w, c_in, c_out,
                            g):
    hc = h * c_in
    kwid = (h + 2) * c_in
    xws = []
    for img in range(g):
        xi = x_ref[img * hc:(img + 1) * hc, :]
        zrow = jnp.zeros((c_in, w), jnp.float32)
        xp = jnp.concatenate([zrow, xi, zrow], axis=0)
        zcol = jnp.zeros((kwid, 1), jnp.float32)
        x0 = jnp.concatenate([zcol, xp[:, :w - 1]], axis=1)
        x2 = jnp.concatenate([xp[:, 1:], zcol], axis=1)
        xws.append(jnp.concatenate([x0, xp, x2], axis=0))
    nslab = h // 8
    kslab = 10 * c_in
    for s in range(nslab):
        pieces = []
        for img in range(g):
            for kw in range(3):
                off = kw * kwid + 8 * s * c_in
                pieces.append(xws[img][off:off + kslab, :])
        rhs = jnp.concatenate(
            [jnp.concatenate(pieces[i * 3:(i + 1) * 3], axis=0)
             for i in range(g)], axis=1).astype(jnp.bfloat16)
        y = jnp.dot(w_ref[...], rhs,
                    preferred_element_type=jnp.float32) + b_ref[...]
        m = _mish(y)
        for img in range(g):
            for co in range(c_out):
                o_ref[img * c_out * h + co * h + 8 * s:
                      img * c_out * h + co * h + 8 * s + 8, :] = (
                    m[co * 8:(co + 1) * 8, img * w:(img + 1) * w])


@jax.jit
def _conv_bn_mish(x, weight, bn_gamma, bn_beta, bn_mean, bn_var):
    eps = 1e-5
    n, c_in, h, w = x.shape
    c_out = weight.shape[0]
    g = 2 if n % 2 == 0 else 1

    scale = bn_gamma / jnp.sqrt(bn_var + eps)
    bias = bn_beta - bn_mean * scale
    w_folded = weight * scale[:, None, None, None]

    d = jnp.arange(10)[None, :] - jnp.arange(8)[:, None]
    band = ((d >= 0) & (d < 3)).astype(jnp.float32)
    w5 = w_folded[:, :, jnp.clip(d, 0, 2), :]
    w5 = w5 * band[None, None, :, :, None]
    ws = (w5.transpose(0, 2, 4, 3, 1)
          .reshape(c_out * 8, 3 * 10 * c_in).astype(jnp.bfloat16))
    b_col = jnp.repeat(bias, 8).reshape(c_out * 8, 1)

    xt = x.transpose(0, 2, 1, 3).reshape(n * h * c_in, w)

    out2 = pl.pallas_call(
        functools.partial(_conv3x3_bn_mish_kernel, h=h, w=w, c_in=c_in,
                          c_out=c_out, g=g),
        out_shape=jax.ShapeDtypeStruct((n * c_out * h, w), jnp.float32),
        grid_spec=pltpu.PrefetchScalarGridSpec(
            num_scalar_prefetch=0,
            grid=(n // g,),
            in_specs=[
                pl.BlockSpec((g * h * c_in, w), lambda i: (i, 0)),
                pl.BlockSpec((c_out * 8, 3 * 10 * c_in), lambda i: (0, 0)),
                pl.BlockSpec((c_out * 8, 1), lambda i: (0, 0)),
            ],
            out_specs=pl.BlockSpec((g * c_out * h, w), lambda i: (i, 0)),
        ),
        compiler_params=pltpu.CompilerParams(
            dimension_semantics=("parallel",)),
    )(xt, ws, b_col)

    return out2.reshape(n, c_out, h, w)


def kernel(x, weight, bn_gamma, bn_beta, bn_mean, bn_var):
    return _conv_bn_mish(x, weight, bn_gamma, bn_beta, bn_mean, bn_var)


# speedup vs baseline: 75.5439x; 1.2906x over previous
import functools

import jax
import jax.numpy as jnp
from jax.experimental import pallas as pl
from jax.experimental.pallas import tpu as pltpu


def _mish(y):
    t = jnp.exp(jnp.minimum(y, 20.0))
    a = t * (2.0 + t)
    return y * a / (2.0 + a)


def _conv3x3_bn_mish_kernel(x_ref, w_ref, b_ref, o_ref, *, h, w, c_in, c_out,
                            g):
    hp = h + 8
    kwid = c_in * hp
    z1 = jnp.zeros((1, w), jnp.float32)
    z7 = jnp.zeros((7, w), jnp.float32)
    xws = []
    for img in range(g):
        chans = []
        for ci in range(c_in):
            base = img * c_in * h + ci * h
            chans += [z1, x_ref[base:base + h, :], z7]
        xp = jnp.concatenate(chans, axis=0)
        zcol = jnp.zeros((kwid, 1), jnp.float32)
        x0 = jnp.concatenate([zcol, xp[:, :w - 1]], axis=1)
        x2 = jnp.concatenate([xp[:, 1:], zcol], axis=1)
        xws.append(jnp.concatenate([x0, xp, x2], axis=0))
    nslab = h // 8
    for s in range(nslab):
        pieces = []
        for img in range(g):
            for kw in range(3):
                for ci in range(c_in):
                    off = kw * kwid + ci * hp + 8 * s
                    pieces.append(xws[img][off:off + 16, :])
        npc = 3 * c_in
        rhs = jnp.concatenate(
            [jnp.concatenate(pieces[i * npc:(i + 1) * npc], axis=0)
             for i in range(g)], axis=1).astype(jnp.bfloat16)
        y = jnp.dot(w_ref[...], rhs,
                    preferred_element_type=jnp.float32) + b_ref[...]
        m = _mish(y)
        for img in range(g):
            for co in range(c_out):
                o_ref[img * c_out * h + co * h + 8 * s:
                      img * c_out * h + co * h + 8 * s + 8, :] = (
                    m[co * 8:(co + 1) * 8, img * w:(img + 1) * w])


@jax.jit
def _conv_bn_mish(x, weight, bn_gamma, bn_beta, bn_mean, bn_var):
    eps = 1e-5
    n, c_in, h, w = x.shape
    c_out = weight.shape[0]
    g = 2 if n % 2 == 0 else 1

    scale = bn_gamma / jnp.sqrt(bn_var + eps)
    bias = bn_beta - bn_mean * scale
    w_folded = weight * scale[:, None, None, None]

    d = jnp.arange(16)[None, :] - jnp.arange(8)[:, None]
    band = ((d >= 0) & (d < 3)).astype(jnp.float32)
    w5 = w_folded[:, :, jnp.clip(d, 0, 2), :]
    w5 = w5 * band[None, None, :, :, None]
    ws = (w5.transpose(0, 2, 4, 1, 3)
          .reshape(c_out * 8, 3 * c_in * 16).astype(jnp.bfloat16))
    b_col = jnp.repeat(bias, 8).reshape(c_out * 8, 1)

    x2 = x.reshape(n * c_in * h, w)

    out2 = pl.pallas_call(
        functools.partial(_conv3x3_bn_mish_kernel, h=h, w=w, c_in=c_in,
                          c_out=c_out, g=g),
        out_shape=jax.ShapeDtypeStruct((n * c_out * h, w), jnp.float32),
        grid_spec=pltpu.PrefetchScalarGridSpec(
            num_scalar_prefetch=0,
            grid=(n // g,),
            in_specs=[
                pl.BlockSpec((g * c_in * h, w), lambda i: (i, 0)),
                pl.BlockSpec((c_out * 8, 3 * c_in * 16), lambda i: (0, 0)),
                pl.BlockSpec((c_out * 8, 1), lambda i: (0, 0)),
            ],
            out_specs=pl.BlockSpec((g * c_out * h, w), lambda i: (i, 0)),
        ),
        compiler_params=pltpu.CompilerParams(
            dimension_semantics=("parallel",)),
    )(x2, ws, b_col)

    return out2.reshape(n, c_out, h, w)


def kernel(x, weight, bn_gamma, bn_beta, bn_mean, bn_var):
    return _conv_bn_mish(x, weight, bn_gamma, bn_beta, bn_mean, bn_var)


# speedup vs baseline: 85.1408x; 1.1270x over previous
import functools

import jax
import jax.numpy as jnp
from jax.experimental import pallas as pl
from jax.experimental.pallas import tpu as pltpu


def _mish(y):
    t = jnp.exp(jnp.minimum(y, 20.0))
    a = t * (2.0 + t)
    return y * a / (2.0 + a)


def _conv3x3_bn_mish_kernel(x_ref, w_ref, b_ref, o_ref, *, h, w, c_in, c_out,
                            g):
    hp = h + 8
    kwid = c_in * hp
    z1 = jnp.zeros((1, w), jnp.float32)
    z7 = jnp.zeros((7, w), jnp.float32)
    xws = []
    for img in range(g):
        chans = []
        for ci in range(c_in):
            base = img * c_in * h + ci * h
            chans += [z1, x_ref[base:base + h, :], z7]
        xp = jnp.concatenate(chans, axis=0)
        zcol = jnp.zeros((kwid, 1), jnp.float32)
        x0 = jnp.concatenate([zcol, xp[:, :w - 1]], axis=1)
        x2 = jnp.concatenate([xp[:, 1:], zcol], axis=1)
        xws.append(jnp.concatenate([x0, xp, x2], axis=0))
    nslab = h // 8
    for s in range(nslab):
        pieces = []
        for img in range(g):
            for kw in range(3):
                for ci in range(c_in):
                    off = kw * kwid + ci * hp + 8 * s
                    pieces.append(xws[img][off:off + 16, :])
        npc = 3 * c_in
        rhs = jnp.concatenate(
            [jnp.concatenate(pieces[i * npc:(i + 1) * npc], axis=0)
             for i in range(g)], axis=1).astype(jnp.bfloat16)
        y = jnp.dot(w_ref[...], rhs,
                    preferred_element_type=jnp.float32) + b_ref[...]
        m = _mish(y)
        for img in range(g):
            for co in range(c_out):
                o_ref[img * c_out * h + co * h + 8 * s:
                      img * c_out * h + co * h + 8 * s + 8, :] = (
                    m[co * 8:(co + 1) * 8, img * w:(img + 1) * w])


@jax.jit
def _conv_bn_mish(x, weight, bn_gamma, bn_beta, bn_mean, bn_var):
    eps = 1e-5
    n, c_in, h, w = x.shape
    c_out = weight.shape[0]
    g = 4 if n % 4 == 0 else (2 if n % 2 == 0 else 1)

    scale = bn_gamma / jnp.sqrt(bn_var + eps)
    bias = bn_beta - bn_mean * scale
    w_folded = weight * scale[:, None, None, None]

    d = jnp.arange(16)[None, :] - jnp.arange(8)[:, None]
    band = ((d >= 0) & (d < 3)).astype(jnp.float32)
    w5 = w_folded[:, :, jnp.clip(d, 0, 2), :]
    w5 = w5 * band[None, None, :, :, None]
    ws = (w5.transpose(0, 2, 4, 1, 3)
          .reshape(c_out * 8, 3 * c_in * 16).astype(jnp.bfloat16))
    b_col = jnp.repeat(bias, 8).reshape(c_out * 8, 1)

    x2 = x.reshape(n * c_in * h, w)

    out2 = pl.pallas_call(
        functools.partial(_conv3x3_bn_mish_kernel, h=h, w=w, c_in=c_in,
                          c_out=c_out, g=g),
        out_shape=jax.ShapeDtypeStruct((n * c_out * h, w), jnp.float32),
        grid_spec=pltpu.PrefetchScalarGridSpec(
            num_scalar_prefetch=0,
            grid=(n // g,),
            in_specs=[
                pl.BlockSpec((g * c_in * h, w), lambda i: (i, 0)),
                pl.BlockSpec((c_out * 8, 3 * c_in * 16), lambda i: (0, 0)),
                pl.BlockSpec((c_out * 8, 1), lambda i: (0, 0)),
            ],
            out_specs=pl.BlockSpec((g * c_out * h, w), lambda i: (i, 0)),
        ),
        compiler_params=pltpu.CompilerParams(
            dimension_semantics=("parallel",)),
    )(x2, ws, b_col)

    return out2.reshape(n, c_out, h, w)


def kernel(x, weight, bn_gamma, bn_beta, bn_mean, bn_var):
    return _conv_bn_mish(x, weight, bn_gamma, bn_beta, bn_mean, bn_var)
